# Initial kernel scaffold; baseline (speedup 1.0000x reference)
import jax
import jax.numpy as jnp
from jax.experimental import pallas as pl


def kernel(input, sincos_alpha, sincos_beta, sincos_gamma):
    raise NotImplementedError("write your pallas kernel here")


# speedup vs baseline: 6.6582x; 6.6582x over previous
import functools
import numpy as np
import jax
import jax.numpy as jnp
from jax import lax
from jax.experimental import pallas as pl
from jax.experimental.pallas import tpu as pltpu

_LS = [0, 1, 2, 3, 4, 5, 6]
_MAXM = max(_LS)
_DIM = sum(2 * l + 1 for l in _LS)


def _real_basis_u(l):
    d = 2 * l + 1
    U = np.zeros((d, d), dtype=np.complex128)
    U[l, l] = 1.0
    for m in range(1, l + 1):
        U[l + m, l + m] = ((-1.0) ** m) / np.sqrt(2.0)
        U[l + m, l - m] = 1.0 / np.sqrt(2.0)
        U[l - m, l + m] = -1j * ((-1.0) ** m) / np.sqrt(2.0)
        U[l - m, l - m] = 1j / np.sqrt(2.0)
    return U


def _j_matrix(l, theta=-np.pi / 2):
    d = 2 * l + 1
    ms = np.arange(-l, l + 1)
    Lp = np.zeros((d, d))
    for i in range(d - 1):
        mm = ms[i]
        Lp[i + 1, i] = np.sqrt(l * (l + 1) - mm * (mm + 1))
    Lx = (Lp + Lp.T) / 2.0
    w, V = np.linalg.eigh(Lx)
    Dc = (V * np.exp(-1j * theta * w)) @ V.conj().T
    U = _real_basis_u(l)
    return np.real(U @ Dc @ U.conj().T)


def _build_consts():
    Jf = np.zeros((_DIM, _DIM), dtype=np.float64)
    P = np.zeros((_DIM, _DIM), dtype=np.float64)
    mabs = np.zeros(_DIM, dtype=np.int64)
    sgn = np.zeros(_DIM, dtype=np.float64)
    off = 0
    for l in _LS:
        d = 2 * l + 1
        Jf[off:off + d, off:off + d] = _j_matrix(l)
        c = off + l
        for i in range(off, off + d):
            P[i, 2 * c - i] = 1.0
            mabs[i] = abs(i - c)
            sgn[i] = 1.0 if i >= c else -1.0
        off += d
    JT = Jf.T
    Ks = np.stack([Jf, Jf @ P, P @ Jf, P @ Jf @ P, JT, JT @ P], axis=0)
    MC = np.zeros((_MAXM + 1, _DIM))
    MS = np.zeros((_MAXM + 1, _DIM))
    for i in range(_DIM):
        MC[mabs[i], i] = 1.0
        MS[mabs[i], i] = sgn[i]
    M = np.stack([MC, MS], axis=0)
    return Ks.astype(np.float32), M.astype(np.float32)


_KS, _M = _build_consts()


def _coeffs(sc_ref, m_ref):
    sin = sc_ref[:, 0, :]
    cos = sc_ref[:, 1, :]
    C = lax.dot(cos, m_ref[0], preferred_element_type=jnp.float32)
    S = lax.dot(sin, m_ref[1], preferred_element_type=jnp.float32)
    return C, S


def _body(x_ref, sa_ref, sb_ref, sg_ref, k_ref, m_ref, o_ref):
    Ca, Sa = _coeffs(sa_ref, m_ref)
    Cb, Sb = _coeffs(sb_ref, m_ref)
    Cg, Sg = _coeffs(sg_ref, m_ref)

    J, JP, PJ, PJP, JT, JTP = (k_ref[i] for i in range(6))
    inner = Cb[:, None, :] * J - Sb[:, None, :] * JP
    inner2 = Cb[:, None, :] * PJ - Sb[:, None, :] * PJP
    L = Ca[:, :, None] * inner + Sa[:, :, None] * inner2
    R = Cg[:, None, :] * JT - Sg[:, None, :] * JTP

    D = lax.dot_general(L, R, (((2,), (1,)), ((0,), (0,))),
                        preferred_element_type=jnp.float32)
    y = lax.dot_general(D, x_ref[...], (((2,), (1,)), ((0,), (0,))),
                        preferred_element_type=jnp.float32)
    o_ref[...] = y


@jax.jit
def kernel(input, sincos_alpha, sincos_beta, sincos_gamma):
    B, dim, C = input.shape
    TB = 128
    grid = (B // TB,)
    sc_spec = pl.BlockSpec((TB, 2, _MAXM + 1), lambda i: (i, 0, 0))
    return pl.pallas_call(
        _body,
        grid=grid,
        in_specs=[
            pl.BlockSpec((TB, dim, C), lambda i: (i, 0, 0)),
            sc_spec, sc_spec, sc_spec,
            pl.BlockSpec((6, _DIM, _DIM), lambda i: (0, 0, 0)),
            pl.BlockSpec((2, _MAXM + 1, _DIM), lambda i: (0, 0, 0)),
        ],
        out_specs=pl.BlockSpec((TB, dim, C), lambda i: (i, 0, 0)),
        out_shape=jax.ShapeDtypeStruct((B, dim, C), input.dtype),
    )(input, sincos_alpha, sincos_beta, sincos_gamma,
      jnp.asarray(_KS), jnp.asarray(_M))


# speedup vs baseline: 8.7344x; 1.3118x over previous
import numpy as np
import jax
import jax.numpy as jnp
from jax.experimental import pallas as pl

_LS = [0, 1, 2, 3, 4, 5, 6]
_MAXM = max(_LS)
_DIM = sum(2 * l + 1 for l in _LS)


def _real_basis_u(l):
    d = 2 * l + 1
    U = np.zeros((d, d), dtype=np.complex128)
    U[l, l] = 1.0
    for m in range(1, l + 1):
        U[l + m, l + m] = ((-1.0) ** m) / np.sqrt(2.0)
        U[l + m, l - m] = 1.0 / np.sqrt(2.0)
        U[l - m, l + m] = -1j * ((-1.0) ** m) / np.sqrt(2.0)
        U[l - m, l - m] = 1j / np.sqrt(2.0)
    return U


def _j_matrix(l, theta=-np.pi / 2):
    d = 2 * l + 1
    ms = np.arange(-l, l + 1)
    Lp = np.zeros((d, d))
    for i in range(d - 1):
        mm = ms[i]
        Lp[i + 1, i] = np.sqrt(l * (l + 1) - mm * (mm + 1))
    Lx = (Lp + Lp.T) / 2.0
    w, V = np.linalg.eigh(Lx)
    Dc = (V * np.exp(-1j * theta * w)) @ V.conj().T
    U = _real_basis_u(l)
    return np.real(U @ Dc @ U.conj().T)


_J_NP = [np.asarray(_j_matrix(l), dtype=np.float32) for l in _LS]


def _z_apply(rows, l, cosv, sinv):
    d = 2 * l + 1
    out = []
    for k in range(d):
        m = k - l
        if m == 0:
            out.append(rows[k])
        elif m > 0:
            out.append(cosv[m] * rows[k] + sinv[m] * rows[2 * l - k])
        else:
            out.append(cosv[-m] * rows[k] - sinv[-m] * rows[2 * l - k])
    return out


def _j_apply(rows, Jm):
    d = Jm.shape[0]
    out = []
    for i in range(d):
        acc = None
        for j in range(d):
            v = float(Jm[i, j])
            if v == 0.0:
                continue
            term = rows[j] if v == 1.0 else v * rows[j]
            acc = term if acc is None else acc + term
        out.append(acc)
    return out


def _coeff_tiles(sc_ref, C):
    cosv, sinv = {}, {}
    for m in range(1, _MAXM + 1):
        cv = sc_ref[m, 1, :]
        sv = sc_ref[m, 0, :]
        cosv[m] = jnp.broadcast_to(cv[:, None], (cv.shape[0], C))
        sinv[m] = jnp.broadcast_to(sv[:, None], (sv.shape[0], C))
    return cosv, sinv


def _body(x_ref, sa_ref, sb_ref, sg_ref, o_ref):
    C = x_ref.shape[-1]
    ca, sa = _coeff_tiles(sa_ref, C)
    cb, sb = _coeff_tiles(sb_ref, C)
    cg, sg = _coeff_tiles(sg_ref, C)
    for l in _LS:
        d = 2 * l + 1
        off = l * l
        rows = [x_ref[off + k] for k in range(d)]
        t = _z_apply(rows, l, cg, sg)
        t = _j_apply(t, _J_NP[l].T)
        t = _z_apply(t, l, cb, sb)
        t = _j_apply(t, _J_NP[l])
        t = _z_apply(t, l, ca, sa)
        for k in range(d):
            o_ref[off + k] = t[k]


@jax.jit
def kernel(input, sincos_alpha, sincos_beta, sincos_gamma):
    B, dim, C = input.shape
    TB = 128
    xt = jnp.transpose(input, (1, 0, 2))
    scs = [jnp.transpose(s, (2, 1, 0))
           for s in (sincos_alpha, sincos_beta, sincos_gamma)]
    sc_spec = pl.BlockSpec((_MAXM + 1, 2, TB), lambda i: (0, 0, i))
    yt = pl.pallas_call(
        _body,
        grid=(B // TB,),
        in_specs=[
            pl.BlockSpec((dim, TB, C), lambda i: (0, i, 0)),
            sc_spec, sc_spec, sc_spec,
        ],
        out_specs=pl.BlockSpec((dim, TB, C), lambda i: (0, i, 0)),
        out_shape=jax.ShapeDtypeStruct((dim, B, C), input.dtype),
    )(xt, *scs)
    return jnp.transpose(yt, (1, 0, 2))


# speedup vs baseline: 19.4667x; 2.2287x over previous
import numpy as np
import jax
import jax.numpy as jnp
from jax.experimental import pallas as pl

_LS = [0, 1, 2, 3, 4, 5, 6]
_MAXM = max(_LS)
_DIM = sum(2 * l + 1 for l in _LS)


def _real_basis_u(l):
    d = 2 * l + 1
    U = np.zeros((d, d), dtype=np.complex128)
    U[l, l] = 1.0
    for m in range(1, l + 1):
        U[l + m, l + m] = ((-1.0) ** m) / np.sqrt(2.0)
        U[l + m, l - m] = 1.0 / np.sqrt(2.0)
        U[l - m, l + m] = -1j * ((-1.0) ** m) / np.sqrt(2.0)
        U[l - m, l - m] = 1j / np.sqrt(2.0)
    return U


def _j_matrix(l, theta=-np.pi / 2):
    d = 2 * l + 1
    ms = np.arange(-l, l + 1)
    Lp = np.zeros((d, d))
    for i in range(d - 1):
        mm = ms[i]
        Lp[i + 1, i] = np.sqrt(l * (l + 1) - mm * (mm + 1))
    Lx = (Lp + Lp.T) / 2.0
    w, V = np.linalg.eigh(Lx)
    Dc = (V * np.exp(-1j * theta * w)) @ V.conj().T
    U = _real_basis_u(l)
    return np.real(U @ Dc @ U.conj().T)


def _clean(J):
    J = np.where(np.abs(J) < 1e-6, 0.0, J)
    J = np.where(np.abs(J - 1.0) < 1e-6, 1.0, J)
    J = np.where(np.abs(J + 1.0) < 1e-6, -1.0, J)
    return np.asarray(J, dtype=np.float32)


_J_NP = [_clean(_j_matrix(l)) for l in _LS]


def _z_apply(rows, l, cosv, sinv):
    d = 2 * l + 1
    out = []
    for k in range(d):
        m = k - l
        if m == 0:
            out.append(rows[k])
        elif m > 0:
            out.append(cosv[m] * rows[k] + sinv[m] * rows[2 * l - k])
        else:
            out.append(cosv[-m] * rows[k] - sinv[-m] * rows[2 * l - k])
    return out


def _j_apply(rows, Jm):
    d = Jm.shape[0]
    out = []
    for i in range(d):
        acc = None
        for j in range(d):
            v = float(Jm[i, j])
            if v == 0.0:
                continue
            term = rows[j] if v == 1.0 else v * rows[j]
            acc = term if acc is None else acc + term
        out.append(acc)
    return out


def _coeff_tiles(sc_ref, C):
    cosv, sinv = {}, {}
    for m in range(1, _MAXM + 1):
        cv = sc_ref[m, 1, :]
        sv = sc_ref[m, 0, :]
        cosv[m] = jnp.broadcast_to(cv[:, None], (cv.shape[0], C))
        sinv[m] = jnp.broadcast_to(sv[:, None], (sv.shape[0], C))
    return cosv, sinv


def _body(x_ref, sa_ref, sb_ref, sg_ref, o_ref):
    C = x_ref.shape[-1]
    ca, sa = _coeff_tiles(sa_ref, C)
    cb, sb = _coeff_tiles(sb_ref, C)
    cg, sg = _coeff_tiles(sg_ref, C)
    for l in _LS:
        d = 2 * l + 1
        off = l * l
        rows = [x_ref[off + k] for k in range(d)]
        t = _z_apply(rows, l, cg, sg)
        t = _j_apply(t, _J_NP[l].T)
        t = _z_apply(t, l, cb, sb)
        t = _j_apply(t, _J_NP[l])
        t = _z_apply(t, l, ca, sa)
        for k in range(d):
            o_ref[off + k] = t[k]


@jax.jit
def kernel(input, sincos_alpha, sincos_beta, sincos_gamma):
    B, dim, C = input.shape
    TB = 128
    xt = jnp.transpose(input, (1, 0, 2))
    scs = [jnp.transpose(s, (2, 1, 0))
           for s in (sincos_alpha, sincos_beta, sincos_gamma)]
    sc_spec = pl.BlockSpec((_MAXM + 1, 2, TB), lambda i: (0, 0, i))
    yt = pl.pallas_call(
        _body,
        grid=(B // TB,),
        in_specs=[
            pl.BlockSpec((dim, TB, C), lambda i: (0, i, 0)),
            sc_spec, sc_spec, sc_spec,
        ],
        out_specs=pl.BlockSpec((dim, TB, C), lambda i: (0, i, 0)),
        out_shape=jax.ShapeDtypeStruct((dim, B, C), input.dtype),
    )(xt, *scs)
    return jnp.transpose(yt, (1, 0, 2))


# speedup vs baseline: 20.4020x; 1.0480x over previous
import numpy as np
import jax
import jax.numpy as jnp
from jax.experimental import pallas as pl

_LS = [0, 1, 2, 3, 4, 5, 6]
_MAXM = max(_LS)
_DIM = sum(2 * l + 1 for l in _LS)


def _real_basis_u(l):
    d = 2 * l + 1
    U = np.zeros((d, d), dtype=np.complex128)
    U[l, l] = 1.0
    for m in range(1, l + 1):
        U[l + m, l + m] = ((-1.0) ** m) / np.sqrt(2.0)
        U[l + m, l - m] = 1.0 / np.sqrt(2.0)
        U[l - m, l + m] = -1j * ((-1.0) ** m) / np.sqrt(2.0)
        U[l - m, l - m] = 1j / np.sqrt(2.0)
    return U


def _j_matrix(l, theta=-np.pi / 2):
    d = 2 * l + 1
    ms = np.arange(-l, l + 1)
    Lp = np.zeros((d, d))
    for i in range(d - 1):
        mm = ms[i]
        Lp[i + 1, i] = np.sqrt(l * (l + 1) - mm * (mm + 1))
    Lx = (Lp + Lp.T) / 2.0
    w, V = np.linalg.eigh(Lx)
    Dc = (V * np.exp(-1j * theta * w)) @ V.conj().T
    U = _real_basis_u(l)
    return np.real(U @ Dc @ U.conj().T)


def _clean(J):
    J = np.where(np.abs(J) < 1e-6, 0.0, J)
    J = np.where(np.abs(J - 1.0) < 1e-6, 1.0, J)
    J = np.where(np.abs(J + 1.0) < 1e-6, -1.0, J)
    return np.asarray(J, dtype=np.float32)


_J_NP = [_clean(_j_matrix(l)) for l in _LS]


def _z_apply(rows, l, cosv, sinv):
    d = 2 * l + 1
    out = []
    for k in range(d):
        m = k - l
        if m == 0:
            out.append(rows[k])
        elif m > 0:
            out.append(cosv[m] * rows[k] + sinv[m] * rows[2 * l - k])
        else:
            out.append(cosv[-m] * rows[k] - sinv[-m] * rows[2 * l - k])
    return out


def _j_apply(rows, Jm):
    d = Jm.shape[0]
    out = []
    for i in range(d):
        acc = None
        for j in range(d):
            v = float(Jm[i, j])
            if v == 0.0:
                continue
            term = rows[j] if v == 1.0 else v * rows[j]
            acc = term if acc is None else acc + term
        out.append(acc)
    return out


def _coeff_tiles(sc_ref, C):
    cosv, sinv = {}, {}
    for m in range(1, _MAXM + 1):
        cv = sc_ref[m, 1, :]
        sv = sc_ref[m, 0, :]
        cosv[m] = jnp.broadcast_to(cv[:, None], (cv.shape[0], C))
        sinv[m] = jnp.broadcast_to(sv[:, None], (sv.shape[0], C))
    return cosv, sinv


def _body(x_ref, sa_ref, sb_ref, sg_ref, o_ref):
    C = x_ref.shape[-1]
    ca, sa = _coeff_tiles(sa_ref, C)
    cb, sb = _coeff_tiles(sb_ref, C)
    cg, sg = _coeff_tiles(sg_ref, C)
    for l in _LS:
        d = 2 * l + 1
        off = l * l
        rows = [x_ref[off + k] for k in range(d)]
        t = _z_apply(rows, l, cg, sg)
        t = _j_apply(t, _J_NP[l].T)
        t = _z_apply(t, l, cb, sb)
        t = _j_apply(t, _J_NP[l])
        t = _z_apply(t, l, ca, sa)
        for k in range(d):
            o_ref[off + k] = t[k]


@jax.jit
def kernel(input, sincos_alpha, sincos_beta, sincos_gamma):
    B, dim, C = input.shape
    TB = 256
    xt = jnp.transpose(input, (1, 0, 2))
    scs = [jnp.transpose(s, (2, 1, 0))
           for s in (sincos_alpha, sincos_beta, sincos_gamma)]
    sc_spec = pl.BlockSpec((_MAXM + 1, 2, TB), lambda i: (0, 0, i))
    yt = pl.pallas_call(
        _body,
        grid=(B // TB,),
        in_specs=[
            pl.BlockSpec((dim, TB, C), lambda i: (0, i, 0)),
            sc_spec, sc_spec, sc_spec,
        ],
        out_specs=pl.BlockSpec((dim, TB, C), lambda i: (0, i, 0)),
        out_shape=jax.ShapeDtypeStruct((dim, B, C), input.dtype),
    )(xt, *scs)
    return jnp.transpose(yt, (1, 0, 2))
